# Initial kernel scaffold; baseline (speedup 1.0000x reference)
import jax
import jax.numpy as jnp
from jax.experimental import pallas as pl


def kernel(idx, table):
    raise NotImplementedError("write your pallas kernel here")


# speedup vs baseline: 1.3306x; 1.3306x over previous
import functools

import jax
import jax.numpy as jnp
from jax import lax
from jax.experimental import pallas as pl
from jax.experimental.pallas import tpu as pltpu
from jax.experimental.pallas import tpu_sc as plsc

_N_VOCAB = 1000
_D = 1000
_B = 1024
_T = 20
_B_TOTAL = _B * _T
_NW = 32
_B_PER_W = _B_TOTAL // _NW
_CHUNK = 64
_N_CHUNKS = _B_PER_W // _CHUNK


@functools.partial(
    pl.kernel,
    mesh=plsc.VectorSubcoreMesh(core_axis_name="c", subcore_axis_name="s"),
    out_type=jax.ShapeDtypeStruct((_B_TOTAL, _D), jnp.float32),
    compiler_params=pltpu.CompilerParams(use_tc_tiling_on_sc=False),
    scratch_types=[
        pltpu.VMEM((_N_CHUNKS, _CHUNK), jnp.int32),
        pltpu.VMEM((_CHUNK, _D), jnp.float32),
        pltpu.SemaphoreType.DMA,
    ],
)
def _gather_rows(idx_hbm, table_hbm, out_hbm, idx_v, rows_v, sem):
    wid = lax.axis_index("s") * 2 + lax.axis_index("c")
    base = wid * _B_PER_W
    pltpu.sync_copy(idx_hbm.at[wid], idx_v)
    for c in range(_N_CHUNKS):
        pltpu.async_copy(table_hbm.at[idx_v.at[c]], rows_v, sem).wait()
        pltpu.sync_copy(rows_v, out_hbm.at[pl.ds(base + c * _CHUNK, _CHUNK)])


def kernel(idx, table):
    idx_r = idx.reshape(_NW, _N_CHUNKS, _CHUNK)
    out = _gather_rows(idx_r, table)
    return out.reshape(_B, _T, _N_VOCAB)


# speedup vs baseline: 1.3588x; 1.0212x over previous
import functools

import jax
import jax.numpy as jnp
from jax import lax
from jax.experimental import pallas as pl
from jax.experimental.pallas import tpu as pltpu
from jax.experimental.pallas import tpu_sc as plsc

_N_VOCAB = 1000
_D = 1000
_B = 1024
_T = 20
_B_TOTAL = _B * _T
_NW = 32
_B_PER_W = _B_TOTAL // _NW
_CHUNK = 64
_N_CHUNKS = _B_PER_W // _CHUNK


@functools.partial(
    pl.kernel,
    mesh=plsc.VectorSubcoreMesh(core_axis_name="c", subcore_axis_name="s"),
    out_type=jax.ShapeDtypeStruct((_B_TOTAL, _D), jnp.float32),
    compiler_params=pltpu.CompilerParams(use_tc_tiling_on_sc=False),
    scratch_types=[
        pltpu.VMEM((_N_CHUNKS, _CHUNK), jnp.int32),
        pltpu.VMEM((2, _CHUNK, _D), jnp.float32),
        pltpu.SemaphoreType.DMA,
        pltpu.SemaphoreType.DMA,
        pltpu.SemaphoreType.DMA,
        pltpu.SemaphoreType.DMA,
    ],
)
def _gather_rows(idx_hbm, table_hbm, out_hbm, idx_v, rows_v, gs0, gs1, ss0, ss1):
    wid = lax.axis_index("s") * 2 + lax.axis_index("c")
    base = wid * _B_PER_W
    pltpu.sync_copy(idx_hbm.at[wid], idx_v)
    gsem = [gs0, gs1]
    ssem = [ss0, ss1]
    gcp = [None, None]
    scp = [None, None]
    gcp[0] = pltpu.async_copy(table_hbm.at[idx_v.at[0]], rows_v.at[0], gsem[0])
    for c in range(_N_CHUNKS):
        b = c % 2
        nb = (c + 1) % 2
        if c + 1 < _N_CHUNKS:
            if scp[nb] is not None:
                scp[nb].wait()
            gcp[nb] = pltpu.async_copy(
                table_hbm.at[idx_v.at[c + 1]], rows_v.at[nb], gsem[nb]
            )
        gcp[b].wait()
        scp[b] = pltpu.async_copy(
            rows_v.at[b], out_hbm.at[pl.ds(base + c * _CHUNK, _CHUNK)], ssem[b]
        )
    scp[0].wait()
    scp[1].wait()


def kernel(idx, table):
    idx_r = idx.reshape(_NW, _N_CHUNKS, _CHUNK)
    out = _gather_rows(idx_r, table)
    return out.reshape(_B, _T, _N_VOCAB)
